# Initial kernel scaffold; baseline (speedup 1.0000x reference)
import jax
import jax.numpy as jnp
from jax.experimental import pallas as pl


def kernel(x, edge_index, edge_attr, x_emb_W, edge_W, edge_b, W1, b1, W2, b2, bn_gamma, bn_beta):
    raise NotImplementedError("write your pallas kernel here")


# speedup vs baseline: 2.1915x; 2.1915x over previous
import functools

import jax
import jax.numpy as jnp
from jax import lax
from jax.experimental import pallas as pl
from jax.experimental.pallas import tpu as pltpu
from jax.experimental.pallas import tpu_sc as plsc

NC = 2
NS = 16
NW = NC * NS

CHUNK = 128
BLK = 64
OWN = 648
REAL = 640
N_PAD = NS * REAL
ACC_ROWS = NS * OWN
CAP = 11264
ZR = 72

F32 = jnp.float32
I32 = jnp.int32


def _sc_route(src3d, dst3d):
    n_chunks = src3d.shape[1]
    per_w = n_chunks * CHUNK
    n_vregs = per_w // 16
    e_pad = per_w * NW
    ring = 8

    def body(src_h, dst_h, psrc_h, pdst_h, pei_h, cnt_h,
             src_v, dst_v, val_v, cnt_v, sem):
        c = lax.axis_index("c")
        s = lax.axis_index("s")
        w = s * NC + c
        pltpu.sync_copy(src_h.at[w], src_v)
        pltpu.sync_copy(dst_h.at[w], dst_v)
        iota = lax.iota(I32, 16)

        def gat(x, idx):
            return x.at[idx].get(mode="promise_in_bounds")

        def drain(n):
            for _ in range(n):
                pltpu.make_async_copy(psrc_h.at[0].at[pl.ds(0, 16)],
                                      val_v.at[0].at[0], sem).wait()

        def step(v, offs):
            row = v // 8
            q = v % 8
            dstv = dst_v.at[row][pl.ds(q * 16, 16)]
            srcv = src_v.at[row][pl.ds(q * 16, 16)]
            peiv = w * per_w + v * 16 + iota
            bk = lax.shift_right_logical(dstv * 6554, 22)
            pd = dstv + 8 * bk
            rank = jnp.zeros((16,), I32)
            for d in range(1, 16):
                same = gat(bk, jnp.maximum(iota - d, 0)) == bk
                rank = rank + jnp.where((iota >= d) & same, 1, 0)
            pos = bk * (NS * CAP) + s * CAP + gat(offs, bk) + rank
            slot = v % ring
            val_v.at[slot].at[0][pl.ds(0, 16)] = srcv
            val_v.at[slot].at[1][pl.ds(0, 16)] = pd
            val_v.at[slot].at[2][pl.ds(0, 16)] = peiv
            pltpu.async_copy(val_v.at[slot].at[0], psrc_h.at[c].at[pos], sem)
            pltpu.async_copy(val_v.at[slot].at[1], pdst_h.at[c].at[pos], sem)
            pltpu.async_copy(val_v.at[slot].at[2], pei_h.at[c].at[pos], sem)
            drain(3)

            upd = jnp.zeros((16,), I32)
            for b in range(NS):
                sel = jnp.where(bk == b, rank + 1, 0)
                for d in (1, 2, 4, 8):
                    sel = jnp.maximum(sel, gat(sel, iota ^ d))
                upd = upd + jnp.where(iota == b, sel, 0)
            return offs + upd

        offs = lax.fori_loop(0, n_vregs, step, jnp.zeros((16,), I32))

        for b in range(NS):
            off_b = gat(offs, iota * 0 + b)
            base = b * (NS * CAP) + s * CAP
            pad_src = (iota * 11 + b * 13) % 9984 + 8
            pad_dst = b * OWN + REAL + (iota % 8)
            pad_pei = e_pad - 16 + iota
            for t in range(4):
                slot = t % 4
                pos = base + off_b + t * 16 + iota
                val_v.at[slot].at[0][pl.ds(0, 16)] = pad_src
                val_v.at[slot].at[1][pl.ds(0, 16)] = pad_dst
                val_v.at[slot].at[2][pl.ds(0, 16)] = pad_pei
                pltpu.async_copy(val_v.at[slot].at[0], psrc_h.at[c].at[pos], sem)
                pltpu.async_copy(val_v.at[slot].at[1], pdst_h.at[c].at[pos], sem)
                pltpu.async_copy(val_v.at[slot].at[2], pei_h.at[c].at[pos], sem)
                drain(3)

        cnt_v[...] = lax.shift_right_logical(offs + (BLK - 1), 6)
        pltpu.sync_copy(cnt_v, cnt_h.at[c, s])

    return pl.kernel(
        body,
        out_type=(
            jax.ShapeDtypeStruct((NC, NS * NS * CAP), I32),
            jax.ShapeDtypeStruct((NC, NS * NS * CAP), I32),
            jax.ShapeDtypeStruct((NC, NS * NS * CAP), I32),
            jax.ShapeDtypeStruct((NC, NS, NS), I32),
        ),
        mesh=plsc.VectorSubcoreMesh(core_axis_name="c", subcore_axis_name="s"),
        compiler_params=pltpu.CompilerParams(use_tc_tiling_on_sc=False),
        scratch_types=[
            pltpu.VMEM((n_chunks, CHUNK), I32),
            pltpu.VMEM((n_chunks, CHUNK), I32),
            pltpu.VMEM((8, 3, 16), I32),
            pltpu.VMEM((16,), I32),
            pltpu.SemaphoreType.DMA,
        ],
    )(src3d, dst3d)


def _sc_aggr(table, idxg, pdst, cnt, zrows, *, width):
    def body(table_h, idxg_h, pdst_h, cnt_h, z_h, out_h,
             isrc_v, idst_v, rows_v, zbuf, cnt_v, acc, sem0):
        c = lax.axis_index("c")
        s = lax.axis_index("s")
        pltpu.sync_copy(z_h, zbuf)
        for t in range(OWN // ZR):
            pltpu.sync_copy(zbuf, acc.at[pl.ds(s * OWN + t * ZR, ZR)])
        pltpu.sync_copy(cnt_h.at[c], cnt_v)
        cvec = cnt_v.at[s][pl.ds(0, 16)]

        for t2 in range(NS):
            nb = cvec[t2]
            base = t2 * CAP

            def blk(k, _):
                off = base + k * BLK
                pltpu.sync_copy(idxg_h.at[c, s, pl.ds(off, BLK)], isrc_v)
                pltpu.sync_copy(pdst_h.at[c, s, pl.ds(off, BLK)], idst_v)
                pltpu.async_copy(table_h.at[isrc_v], rows_v, sem0).wait()
                pltpu.sync_copy(rows_v, acc.at[idst_v], add=True)
                return 0

            lax.fori_loop(0, nb, blk, 0)

        pltpu.sync_copy(acc.at[pl.ds(s * OWN, OWN)],
                        out_h.at[c, pl.ds(s * OWN, OWN)])

    return pl.kernel(
        body,
        out_type=jax.ShapeDtypeStruct((NC, ACC_ROWS, width), F32),
        mesh=plsc.VectorSubcoreMesh(core_axis_name="c", subcore_axis_name="s"),
        scratch_types=[
            pltpu.VMEM((BLK,), I32),
            pltpu.VMEM((BLK,), I32),
            pltpu.VMEM((BLK, width), F32),
            pltpu.VMEM((ZR, width), F32),
            pltpu.VMEM((NS, 16), I32),
            pltpu.VMEM_SHARED((ACC_ROWS, width), F32),
            pltpu.SemaphoreType.DMA,
        ],
    )(table, idxg, pdst, cnt, zrows)


_PREC = lax.Precision.HIGHEST


def _embed_body(x_r, w_r, out_r):
    out_r[...] = jnp.dot(x_r[...], w_r[...], preferred_element_type=F32,
                         precision=_PREC)


def _tc_embed(x, w_t):
    n = x.shape[0]
    return pl.pallas_call(
        _embed_body,
        out_shape=jax.ShapeDtypeStruct((n, w_t.shape[1]), F32),
    )(x, w_t)


def _softplus(v):
    return jnp.maximum(v, 0.0) + jnp.log1p(jnp.exp(-jnp.abs(v)))


def _gather_blocks(ref, n):
    a = jnp.concatenate([ref[0, b * OWN:b * OWN + REAL] for b in range(NS)],
                        axis=0)
    b_ = jnp.concatenate([ref[1, b * OWN:b * OWN + REAL] for b in range(NS)],
                         axis=0)
    return (a + b_)[:n]


def _layer_body(do_mish, n, p_r, h_r, a_r, we_r, eb_r, w1_r, b1_r,
                w2_r, b2_r, g_r, be_r, out_r):
    apad = _gather_blocks(a_r, n)
    psum = _gather_blocks(p_r, n)
    aggr = (psum + h_r[...] + eb_r[...]
            + jnp.dot(apad, we_r[...], preferred_element_type=F32, precision=_PREC))
    t = jnp.maximum(
        jnp.dot(aggr, w1_r[...], preferred_element_type=F32, precision=_PREC)
        + b1_r[...], 0.0)
    hh = (jnp.dot(t, w2_r[...], preferred_element_type=F32, precision=_PREC)
          + b2_r[...])
    hh = hh * g_r[...] + be_r[...]
    if do_mish:
        hh = hh * jnp.tanh(_softplus(hh))
    out_r[...] = hh


def _tc_layer(parts, h, apad, we, eb, w1t, b1, w2t, b2, g, be, do_mish):
    n, emb = h.shape
    return pl.pallas_call(
        functools.partial(_layer_body, do_mish, n),
        out_shape=jax.ShapeDtypeStruct((n, emb), F32),
    )(parts, h, apad, we, eb[None, :], w1t, b1[None, :], w2t, b2[None, :],
      g[None, :], be[None, :])


def kernel(x, edge_index, edge_attr, x_emb_W, edge_W, edge_b, W1, b1, W2, b2,
           bn_gamma, bn_beta):
    n_nodes, atom = x.shape
    n_edges = edge_index.shape[1]
    bond = edge_attr.shape[1]
    emb = x_emb_W.shape[0]
    num_layer = W1.shape[0]

    e_pad = -(-n_edges // (NW * CHUNK)) * NW * CHUNK
    n_chunks = e_pad // (NW * CHUNK)
    npd = e_pad - n_edges
    pad_src = (jnp.arange(npd, dtype=I32) * 7) % n_nodes
    pad_dst = n_nodes + jnp.arange(npd, dtype=I32) % (N_PAD - n_nodes)
    src3d = jnp.concatenate([edge_index[0], pad_src]).reshape(NW, n_chunks, CHUNK)
    dst3d = jnp.concatenate([edge_index[1], pad_dst]).reshape(NW, n_chunks, CHUNK)

    wa = 128
    ea_pad = jnp.concatenate(
        [jnp.concatenate(
            [edge_attr, jnp.ones((n_edges, 1), F32),
             jnp.zeros((n_edges, wa - bond - 1), F32)], axis=1),
         jnp.zeros((npd, wa), F32)], axis=0)

    zrows = jnp.zeros((ZR, emb), F32)
    zrows_a = jnp.zeros((ZR, wa), F32)

    h = _tc_embed(x, x_emb_W.T)

    psrc, pdst, pei, cnt = _sc_route(src3d, dst3d)
    psrc = psrc.reshape(NC, NS, NS * CAP)
    pdst = pdst.reshape(NC, NS, NS * CAP)
    pei = pei.reshape(NC, NS, NS * CAP)
    cnt_t = jnp.swapaxes(cnt, 1, 2)

    apad = _sc_aggr(ea_pad, pei, pdst, cnt_t, zrows_a, width=wa)

    for l in range(num_layer):
        parts = _sc_aggr(h, psrc, pdst, cnt_t, zrows, width=emb)
        we = jnp.concatenate(
            [edge_W[l].T, edge_b[l][None, :],
             jnp.zeros((wa - bond - 1, emb), F32)], axis=0)
        h = _tc_layer(parts, h, apad, we, edge_b[l],
                      W1[l].T, b1[l], W2[l].T, b2[l], bn_gamma[l], bn_beta[l],
                      do_mish=(l < num_layer - 1))
    return h


# speedup vs baseline: 2.4625x; 1.1237x over previous
import functools

import jax
import jax.numpy as jnp
from jax import lax
from jax.experimental import pallas as pl
from jax.experimental.pallas import tpu as pltpu
from jax.experimental.pallas import tpu_sc as plsc

NC = 2
NS = 16
NW = NC * NS

CHUNK = 128
BLK = 128
OWN = 648
REAL = 640
N_PAD = NS * REAL
ACC_ROWS = NS * OWN
CAP = 16384
ZR = 72

F32 = jnp.float32
I32 = jnp.int32


def _sc_route(src3d, dst3d):
    n_chunks = src3d.shape[1]
    per_w = n_chunks * CHUNK
    n_vregs = per_w // 16
    e_pad = per_w * NW
    ring = 8

    def body(src_h, dst_h, psrc_h, pdst_h, pei_h, cnt_h,
             src_v, dst_v, val_v, cnt_v, sem):
        c = lax.axis_index("c")
        s = lax.axis_index("s")
        w = s * NC + c
        pltpu.sync_copy(src_h.at[w], src_v)
        pltpu.sync_copy(dst_h.at[w], dst_v)
        iota = lax.iota(I32, 16)

        def gat(x, idx):
            return x.at[idx].get(mode="promise_in_bounds")

        def drain(n):
            for _ in range(n):
                pltpu.make_async_copy(psrc_h.at[0].at[pl.ds(0, 16)],
                                      val_v.at[0].at[0], sem).wait()

        def step(v, offs):
            row = v // 8
            q = v % 8
            dstv = dst_v.at[row][pl.ds(q * 16, 16)]
            srcv = src_v.at[row][pl.ds(q * 16, 16)]
            peiv = w * per_w + v * 16 + iota
            bk = lax.shift_right_logical(dstv * 6554, 22)
            pd = dstv + 8 * bk
            rank = jnp.zeros((16,), I32)
            for d in range(1, 16):
                same = gat(bk, jnp.maximum(iota - d, 0)) == bk
                rank = rank + jnp.where((iota >= d) & same, 1, 0)
            pos = bk * (NS * CAP) + s * CAP + gat(offs, bk) + rank
            slot = v % ring
            val_v.at[slot].at[0][pl.ds(0, 16)] = srcv
            val_v.at[slot].at[1][pl.ds(0, 16)] = pd
            val_v.at[slot].at[2][pl.ds(0, 16)] = peiv
            pltpu.async_copy(val_v.at[slot].at[0], psrc_h.at[c].at[pos], sem)
            pltpu.async_copy(val_v.at[slot].at[1], pdst_h.at[c].at[pos], sem)
            pltpu.async_copy(val_v.at[slot].at[2], pei_h.at[c].at[pos], sem)
            drain(3)

            upd = jnp.zeros((16,), I32)
            for b in range(NS):
                sel = jnp.where(bk == b, rank + 1, 0)
                for d in (1, 2, 4, 8):
                    sel = jnp.maximum(sel, gat(sel, iota ^ d))
                upd = upd + jnp.where(iota == b, sel, 0)
            return offs + upd

        offs = lax.fori_loop(0, n_vregs, step, jnp.zeros((16,), I32))

        for b in range(NS):
            off_b = gat(offs, iota * 0 + b)
            base = b * (NS * CAP) + s * CAP
            pad_src = (iota * 11 + b * 13) % 9984 + 8
            pad_dst = b * OWN + REAL + (iota % 8)
            pad_pei = e_pad - 16 + iota
            for t in range(BLK // 16):
                slot = t % 4
                pos = base + off_b + t * 16 + iota
                val_v.at[slot].at[0][pl.ds(0, 16)] = pad_src
                val_v.at[slot].at[1][pl.ds(0, 16)] = pad_dst
                val_v.at[slot].at[2][pl.ds(0, 16)] = pad_pei
                pltpu.async_copy(val_v.at[slot].at[0], psrc_h.at[c].at[pos], sem)
                pltpu.async_copy(val_v.at[slot].at[1], pdst_h.at[c].at[pos], sem)
                pltpu.async_copy(val_v.at[slot].at[2], pei_h.at[c].at[pos], sem)
                drain(3)

        cnt_v[...] = lax.shift_right_logical(offs + (BLK - 1), 7)
        pltpu.sync_copy(cnt_v, cnt_h.at[c, s])

    return pl.kernel(
        body,
        out_type=(
            jax.ShapeDtypeStruct((NC, NS * NS * CAP), I32),
            jax.ShapeDtypeStruct((NC, NS * NS * CAP), I32),
            jax.ShapeDtypeStruct((NC, NS * NS * CAP), I32),
            jax.ShapeDtypeStruct((NC, NS, NS), I32),
        ),
        mesh=plsc.VectorSubcoreMesh(core_axis_name="c", subcore_axis_name="s"),
        compiler_params=pltpu.CompilerParams(use_tc_tiling_on_sc=False),
        scratch_types=[
            pltpu.VMEM((n_chunks, CHUNK), I32),
            pltpu.VMEM((n_chunks, CHUNK), I32),
            pltpu.VMEM((8, 3, 16), I32),
            pltpu.VMEM((16,), I32),
            pltpu.SemaphoreType.DMA,
        ],
    )(src3d, dst3d)


def _sc_aggr(table, idxg, pdst, cnt, zrows, *, width):
    def body(table_h, idxg_h, pdst_h, cnt_h, z_h, out_h,
             isrc_v, idst_v, rows_v, zbuf, cnt_v, acc, sem0, sem1):
        c = lax.axis_index("c")
        s = lax.axis_index("s")
        pltpu.sync_copy(z_h, zbuf)
        for t in range(OWN // ZR):
            pltpu.sync_copy(zbuf, acc.at[pl.ds(s * OWN + t * ZR, ZR)])
        pltpu.sync_copy(cnt_h.at[c], cnt_v)
        cvec = cnt_v.at[s][pl.ds(0, 16)]

        for t2 in range(NS):
            nb = cvec[t2]
            base = t2 * CAP

            def blk(k, _):
                off = base + k * BLK
                pltpu.async_copy(idxg_h.at[c, s, pl.ds(off, BLK)], isrc_v, sem0)
                pltpu.async_copy(pdst_h.at[c, s, pl.ds(off, BLK)], idst_v, sem1)
                pltpu.make_async_copy(idxg_h.at[c, s, pl.ds(off, BLK)], isrc_v,
                                      sem0).wait()
                pltpu.async_copy(table_h.at[isrc_v], rows_v, sem0)
                pltpu.make_async_copy(pdst_h.at[c, s, pl.ds(off, BLK)], idst_v,
                                      sem1).wait()
                pltpu.make_async_copy(table_h.at[isrc_v], rows_v, sem0).wait()
                pltpu.sync_copy(rows_v, acc.at[idst_v], add=True)
                return 0

            lax.fori_loop(0, nb, blk, 0)

        pltpu.sync_copy(acc.at[pl.ds(s * OWN, OWN)],
                        out_h.at[c, pl.ds(s * OWN, OWN)])

    return pl.kernel(
        body,
        out_type=jax.ShapeDtypeStruct((NC, ACC_ROWS, width), F32),
        mesh=plsc.VectorSubcoreMesh(core_axis_name="c", subcore_axis_name="s"),
        scratch_types=[
            pltpu.VMEM((BLK,), I32),
            pltpu.VMEM((BLK,), I32),
            pltpu.VMEM((BLK, width), F32),
            pltpu.VMEM((ZR, width), F32),
            pltpu.VMEM((NS, 16), I32),
            pltpu.VMEM_SHARED((ACC_ROWS, width), F32),
            pltpu.SemaphoreType.DMA,
            pltpu.SemaphoreType.DMA,
        ],
    )(table, idxg, pdst, cnt, zrows)


_PREC = lax.Precision.HIGHEST


def _embed_body(x_r, w_r, out_r):
    out_r[...] = jnp.dot(x_r[...], w_r[...], preferred_element_type=F32,
                         precision=_PREC)


def _tc_embed(x, w_t):
    n = x.shape[0]
    return pl.pallas_call(
        _embed_body,
        out_shape=jax.ShapeDtypeStruct((n, w_t.shape[1]), F32),
    )(x, w_t)


def _softplus(v):
    return jnp.maximum(v, 0.0) + jnp.log1p(jnp.exp(-jnp.abs(v)))


def _gather_blocks(ref, n):
    a = jnp.concatenate([ref[0, b * OWN:b * OWN + REAL] for b in range(NS)],
                        axis=0)
    b_ = jnp.concatenate([ref[1, b * OWN:b * OWN + REAL] for b in range(NS)],
                         axis=0)
    return (a + b_)[:n]


def _layer_body(do_mish, n, p_r, h_r, a_r, we_r, eb_r, w1_r, b1_r,
                w2_r, b2_r, g_r, be_r, out_r):
    apad = _gather_blocks(a_r, n)
    psum = _gather_blocks(p_r, n)
    aggr = (psum + h_r[...] + eb_r[...]
            + jnp.dot(apad, we_r[...], preferred_element_type=F32, precision=_PREC))
    t = jnp.maximum(
        jnp.dot(aggr, w1_r[...], preferred_element_type=F32, precision=_PREC)
        + b1_r[...], 0.0)
    hh = (jnp.dot(t, w2_r[...], preferred_element_type=F32, precision=_PREC)
          + b2_r[...])
    hh = hh * g_r[...] + be_r[...]
    if do_mish:
        hh = hh * jnp.tanh(_softplus(hh))
    out_r[...] = hh


def _tc_layer(parts, h, apad, we, eb, w1t, b1, w2t, b2, g, be, do_mish):
    n, emb = h.shape
    return pl.pallas_call(
        functools.partial(_layer_body, do_mish, n),
        out_shape=jax.ShapeDtypeStruct((n, emb), F32),
    )(parts, h, apad, we, eb[None, :], w1t, b1[None, :], w2t, b2[None, :],
      g[None, :], be[None, :])


def kernel(x, edge_index, edge_attr, x_emb_W, edge_W, edge_b, W1, b1, W2, b2,
           bn_gamma, bn_beta):
    n_nodes, atom = x.shape
    n_edges = edge_index.shape[1]
    bond = edge_attr.shape[1]
    emb = x_emb_W.shape[0]
    num_layer = W1.shape[0]

    e_pad = -(-n_edges // (NW * CHUNK)) * NW * CHUNK
    n_chunks = e_pad // (NW * CHUNK)
    npd = e_pad - n_edges
    pad_src = (jnp.arange(npd, dtype=I32) * 7) % n_nodes
    pad_dst = n_nodes + jnp.arange(npd, dtype=I32) % (N_PAD - n_nodes)
    src3d = jnp.concatenate([edge_index[0], pad_src]).reshape(NW, n_chunks, CHUNK)
    dst3d = jnp.concatenate([edge_index[1], pad_dst]).reshape(NW, n_chunks, CHUNK)

    wa = 128
    ea_pad = jnp.concatenate(
        [jnp.concatenate(
            [edge_attr, jnp.ones((n_edges, 1), F32),
             jnp.zeros((n_edges, wa - bond - 1), F32)], axis=1),
         jnp.zeros((npd, wa), F32)], axis=0)

    zrows = jnp.zeros((ZR, emb), F32)
    zrows_a = jnp.zeros((ZR, wa), F32)

    h = _tc_embed(x, x_emb_W.T)

    psrc, pdst, pei, cnt = _sc_route(src3d, dst3d)
    psrc = psrc.reshape(NC, NS, NS * CAP)
    pdst = pdst.reshape(NC, NS, NS * CAP)
    pei = pei.reshape(NC, NS, NS * CAP)
    cnt_t = jnp.swapaxes(cnt, 1, 2)

    apad = _sc_aggr(ea_pad, pei, pdst, cnt_t, zrows_a, width=wa)

    for l in range(num_layer):
        parts = _sc_aggr(h, psrc, pdst, cnt_t, zrows, width=emb)
        we = jnp.concatenate(
            [edge_W[l].T, edge_b[l][None, :],
             jnp.zeros((wa - bond - 1, emb), F32)], axis=0)
        h = _tc_layer(parts, h, apad, we, edge_b[l],
                      W1[l].T, b1[l], W2[l].T, b2[l], bn_gamma[l], bn_beta[l],
                      do_mish=(l < num_layer - 1))
    return h
